# baseline (speedup 1.0000x reference)
import functools
import math

import jax
import jax.numpy as jnp
from jax.experimental import pallas as pl
from jax.experimental.pallas import tpu as pltpu

_SPARSITY = 0.8


def _reduce_body(x_ref, sum_ref, max_ref, *, F, CB):
    for j in range(CB):
        s = None
        m = None
        for f in range(F):
            blk = x_ref[0, f, j]
            ps = jnp.sum(blk, axis=0)
            pm = jnp.max(blk, axis=0)
            s = ps if s is None else s + ps
            m = pm if m is None else jnp.maximum(m, pm)
        sum_ref[0, j] = s
        max_ref[0, j] = m


def _scale_body(sum_ref, max_ref, w1_ref, w2_ref, scale_ref, *, n_red, k):
    avg = jnp.sum(sum_ref[...], axis=-1) * (1.0 / n_red)
    mx = jnp.max(max_ref[...], axis=-1)
    w1 = w1_ref[...]
    w2 = w2_ref[...]

    def mlp(v):
        h = jnp.sum(v[:, None, :] * w1[None, :, :], axis=-1)
        h = jnp.maximum(h, 0.0)
        return jnp.sum(h[:, None, :] * w2[None, :, :], axis=-1)

    logit = mlp(avg) + mlp(mx)
    ca = 1.0 / (1.0 + jnp.exp(-logit))

    b, c = ca.shape
    sj = ca[:, None, :]
    si = ca[:, :, None]
    ii = jax.lax.broadcasted_iota(jnp.int32, (1, c, c), 1)
    jj = jax.lax.broadcasted_iota(jnp.int32, (1, c, c), 2)
    beats = (sj > si) | ((sj == si) & (jj < ii))
    rank = jnp.sum(beats.astype(jnp.int32), axis=-1)
    scale_ref[...] = jnp.where(rank < k, ca, 0.0)[:, None, :]


def _mul_body(scale_ref, x_ref, out_ref, *, F, CB):
    cb = pl.program_id(1)
    for f in range(F):
        for j in range(CB):
            s = scale_ref[0, 0, cb * CB + j]
            out_ref[0, f, j] = x_ref[0, f, j] * s


def kernel(x, W1, W2):
    B, F, C, H, W = x.shape
    HW = H * W
    LL = 128 if HW % 128 == 0 else 1
    HWG = HW // LL
    CB = 8 if C % 8 == 0 else 1
    NCB = C // CB
    xr = x.reshape(B, F, C, HWG, LL)
    k = int(math.ceil(C * _SPARSITY))

    x_spec = pl.BlockSpec((1, F, CB, HWG, LL), lambda b, cb: (b, 0, cb, 0, 0))
    part_spec = pl.BlockSpec((1, CB, LL), lambda b, cb: (b, cb, 0))

    sums, maxs = pl.pallas_call(
        functools.partial(_reduce_body, F=F, CB=CB),
        grid=(B, NCB),
        in_specs=[x_spec],
        out_specs=[part_spec, part_spec],
        out_shape=[jax.ShapeDtypeStruct((B, C, LL), jnp.float32)] * 2,
        compiler_params=pltpu.CompilerParams(
            dimension_semantics=("parallel", "parallel")),
    )(xr)

    scale = pl.pallas_call(
        functools.partial(_scale_body, n_red=F * HW, k=k),
        out_shape=jax.ShapeDtypeStruct((B, 1, C), jnp.float32),
    )(sums, maxs, W1, W2)

    out = pl.pallas_call(
        functools.partial(_mul_body, F=F, CB=CB),
        grid=(B, NCB),
        in_specs=[
            pl.BlockSpec(memory_space=pltpu.SMEM),
            x_spec,
        ],
        out_specs=x_spec,
        out_shape=jax.ShapeDtypeStruct((B, F, C, HWG, LL), jnp.float32),
        compiler_params=pltpu.CompilerParams(
            dimension_semantics=("parallel", "parallel")),
    )(scale, xr)
    return out.reshape(B, F, C, H, W)
